# baseline (speedup 1.0000x reference)
import functools

import jax
import jax.numpy as jnp
from jax import lax
from jax.experimental import pallas as pl
from jax.experimental.pallas import tpu as pltpu
from jax.experimental.pallas import tpu_sc as plsc

N_USERS = 1000000
N_ITEMS = 100000
K = 90
MU = 3.5
BATCH = 16384

NC = 2
NS = 16
L = 16
NW = NC * NS
B_PER_W = BATCH // NW
IDX_CHUNK = 128
N_IDX_CHUNKS = B_PER_W // IDX_CHUNK


def _lfm_body(uidx_hbm, iidx_hbm, p_hbm, q_hbm, bu_hbm, bi_hbm, out_hbm,
              uidx_v, iidx_v, p_rows, q_rows, bu_v, bi_v, out_v,
              sem_p, sem_q, sem_b):
    wid = lax.axis_index("s") * NC + lax.axis_index("c")
    base = wid * B_PER_W

    for c in range(N_IDX_CHUNKS):
        src = pl.ds(base + c * IDX_CHUNK, IDX_CHUNK)
        pltpu.sync_copy(uidx_hbm.at[src], uidx_v.at[c])
        pltpu.sync_copy(iidx_hbm.at[src], iidx_v.at[c])

    copies = []
    for c in range(N_IDX_CHUNKS):
        sl = pl.ds(c * IDX_CHUNK, IDX_CHUNK)
        copies.append(pltpu.async_copy(p_hbm.at[uidx_v.at[c]], p_rows.at[sl], sem_p))
        copies.append(pltpu.async_copy(q_hbm.at[iidx_v.at[c]], q_rows.at[sl], sem_q))
        copies.append(pltpu.async_copy(bu_hbm.at[uidx_v.at[c]], bu_v.at[sl], sem_b))
        copies.append(pltpu.async_copy(bi_hbm.at[iidx_v.at[c]], bi_v.at[sl], sem_b))
    for cp in copies:
        cp.wait()

    lane = lax.iota(jnp.int32, L)
    tail_mask = lane >= (80 - (K - L))

    def group_dot(g, _):
        row0 = g * L
        vec = jnp.zeros((L,), jnp.float32)
        for j in range(L):
            i = row0 + j
            acc = p_rows[i, pl.ds(0, L)] * q_rows[i, pl.ds(0, L)]
            for c0 in (16, 32, 48, 64):
                acc = acc + p_rows[i, pl.ds(c0, L)] * q_rows[i, pl.ds(c0, L)]
            tail = p_rows[i, pl.ds(K - L, L)] * q_rows[i, pl.ds(K - L, L)]
            acc = acc + jnp.where(tail_mask, tail, 0.0)
            for k in (8, 4, 2, 1):
                acc = acc + jnp.take(acc, lane ^ k)
            vec = jnp.where(lane == j, acc, vec)
        sl = pl.ds(row0, L)
        out_v[sl] = vec + bu_v[sl] + bi_v[sl] + MU
        return 0

    lax.fori_loop(0, B_PER_W // L, group_dot, 0)

    pltpu.sync_copy(out_v, out_hbm.at[pl.ds(base, B_PER_W)])


@jax.jit
def _lfm(user_idx, item_idx, P, Q, bu_flat, bi_flat):
    mesh = plsc.VectorSubcoreMesh(core_axis_name="c", subcore_axis_name="s")
    kern = functools.partial(
        pl.kernel,
        out_type=jax.ShapeDtypeStruct((BATCH,), jnp.float32),
        mesh=mesh,
        compiler_params=pltpu.CompilerParams(use_tc_tiling_on_sc=False),
        scratch_types=[
            pltpu.VMEM((N_IDX_CHUNKS, IDX_CHUNK), jnp.int32),
            pltpu.VMEM((N_IDX_CHUNKS, IDX_CHUNK), jnp.int32),
            pltpu.VMEM((B_PER_W, K), jnp.float32),
            pltpu.VMEM((B_PER_W, K), jnp.float32),
            pltpu.VMEM((B_PER_W,), jnp.float32),
            pltpu.VMEM((B_PER_W,), jnp.float32),
            pltpu.VMEM((B_PER_W,), jnp.float32),
            pltpu.SemaphoreType.DMA,
            pltpu.SemaphoreType.DMA,
            pltpu.SemaphoreType.DMA,
        ],
    )(_lfm_body)
    return kern(user_idx, item_idx, P, Q, bu_flat, bi_flat)


def kernel(user_idx, item_idx, P, Q, b_u, b_i):
    return _lfm(user_idx.astype(jnp.int32), item_idx.astype(jnp.int32),
                P, Q, b_u.reshape(-1), b_i.reshape(-1))


# speedup vs baseline: 4.6913x; 4.6913x over previous
import functools

import jax
import jax.numpy as jnp
from jax import lax
from jax.experimental import pallas as pl
from jax.experimental.pallas import tpu as pltpu
from jax.experimental.pallas import tpu_sc as plsc

K = 90
MU = 3.5
BATCH = 16384

NC = 2
NS = 16
L = 16
NW = NC * NS
B_PER_W = BATCH // NW
IDX_CHUNK = 128
N_IDX_CHUNKS = B_PER_W // IDX_CHUNK
N_GROUPS = B_PER_W // L


def _lfm_body(uidx_hbm, iidx_hbm, p_hbm, q_hbm, bu_hbm, bi_hbm, out_hbm,
              uidx_v, iidx_v, p_rows, q_rows, bu_v, bi_v, out_v,
              sem_p, sem_q, sem_b):
    wid = lax.axis_index("s") * NC + lax.axis_index("c")
    base = wid * B_PER_W

    for c in range(N_IDX_CHUNKS):
        src = pl.ds(base + c * IDX_CHUNK, IDX_CHUNK)
        pltpu.sync_copy(uidx_hbm.at[src], uidx_v.at[c])
        pltpu.sync_copy(iidx_hbm.at[src], iidx_v.at[c])

    bias_copies = []
    for c in range(N_IDX_CHUNKS):
        sl = pl.ds(c * IDX_CHUNK, IDX_CHUNK)
        bias_copies.append(pltpu.async_copy(bu_hbm.at[uidx_v.at[c]], bu_v.at[sl], sem_b))
        bias_copies.append(pltpu.async_copy(bi_hbm.at[iidx_v.at[c]], bi_v.at[sl], sem_b))

    lane = lax.iota(jnp.int32, L)
    tail_mask = lane >= (80 - (K - L))

    def group_step(g, _):
        c = g // (IDX_CHUNK // L)
        o = (g % (IDX_CHUNK // L)) * L
        uvec = uidx_v[c, pl.ds(o, L)]
        ivec = iidx_v[c, pl.ds(o, L)]
        cps = []
        for j in range(L):
            cps.append(pltpu.async_copy(p_hbm.at[uvec[j]], p_rows.at[j], sem_p))
            cps.append(pltpu.async_copy(q_hbm.at[ivec[j]], q_rows.at[j], sem_q))
        for cp in cps:
            cp.wait()
        vec = jnp.zeros((L,), jnp.float32)
        for j in range(L):
            acc = p_rows[j, pl.ds(0, L)] * q_rows[j, pl.ds(0, L)]
            for c0 in (16, 32, 48, 64):
                acc = acc + p_rows[j, pl.ds(c0, L)] * q_rows[j, pl.ds(c0, L)]
            tail = p_rows[j, pl.ds(K - L, L)] * q_rows[j, pl.ds(K - L, L)]
            acc = acc + jnp.where(tail_mask, tail, 0.0)
            for k in (8, 4, 2, 1):
                acc = acc + jnp.take(acc, lane ^ k)
            vec = jnp.where(lane == j, acc, vec)
        out_v[pl.ds(g * L, L)] = vec
        return 0

    lax.fori_loop(0, N_GROUPS, group_step, 0)

    for cp in bias_copies:
        cp.wait()

    def bias_add(c, _):
        sl = pl.ds(c * L, L)
        out_v[sl] = out_v[sl] + bu_v[sl] + bi_v[sl] + MU
        return 0

    lax.fori_loop(0, B_PER_W // L, bias_add, 0, unroll=4)

    pltpu.sync_copy(out_v, out_hbm.at[pl.ds(base, B_PER_W)])


@jax.jit
def _lfm(user_idx, item_idx, P, Q, bu_flat, bi_flat):
    mesh = plsc.VectorSubcoreMesh(core_axis_name="c", subcore_axis_name="s")
    kern = functools.partial(
        pl.kernel,
        out_type=jax.ShapeDtypeStruct((BATCH,), jnp.float32),
        mesh=mesh,
        scratch_types=[
            pltpu.VMEM((N_IDX_CHUNKS, IDX_CHUNK), jnp.int32),
            pltpu.VMEM((N_IDX_CHUNKS, IDX_CHUNK), jnp.int32),
            pltpu.VMEM((L, K), jnp.float32),
            pltpu.VMEM((L, K), jnp.float32),
            pltpu.VMEM((B_PER_W,), jnp.float32),
            pltpu.VMEM((B_PER_W,), jnp.float32),
            pltpu.VMEM((B_PER_W,), jnp.float32),
            pltpu.SemaphoreType.DMA,
            pltpu.SemaphoreType.DMA,
            pltpu.SemaphoreType.DMA,
        ],
    )(_lfm_body)
    return kern(user_idx, item_idx, P, Q, bu_flat, bi_flat)


def kernel(user_idx, item_idx, P, Q, b_u, b_i):
    return _lfm(user_idx.astype(jnp.int32), item_idx.astype(jnp.int32),
                P, Q, b_u.reshape(-1), b_i.reshape(-1))
